# baseline (speedup 1.0000x reference)
import jax
import jax.numpy as jnp
from jax.experimental import pallas as pl
from jax.experimental.pallas import tpu as pltpu

ALPHA = 0.2
N = 1024
NFEAT = 256
NHID = 64
NHEADS = 3
NCLASS = 40
_BNDS = (0, 256, 1024)
NCHUNK = len(_BNDS) - 1

_DOT_T = (((1,), (1,)), ((), ()))


def _elu(x):
    return jnp.where(x >= 0, x, jnp.exp(x) - 1.0)


def _node_exps(h, al, ar):
    M = jnp.concatenate([(1.0 - ALPHA) * al, (1.0 - ALPHA) * ar, -ar],
                        axis=1)
    E = jnp.exp(jnp.dot(h, M, preferred_element_type=jnp.float32))
    b16 = jnp.bfloat16
    return (E[:, 0:1].astype(b16),
            E[:, 1:2].reshape(1, N).astype(b16),
            E[:, 2:3])


def _masked_attn(ex, rows, m16):
    r, E3, _ = ex
    if rows is not None:
        r = r[rows]
    return jnp.minimum(r * E3, m16)


def _body(x_hbm, m_hbm, W0_h, W1_h, W2_h, a0_h, a1_h, a2_h, Wo_h, ao_h,
          out_ref, xv, mv, m16v, Wv, av, Wov, aov, sems):
    cps = [
        pltpu.make_async_copy(x_hbm, xv, sems.at[0]),
        pltpu.make_async_copy(W0_h, Wv.at[0], sems.at[1]),
        pltpu.make_async_copy(W1_h, Wv.at[1], sems.at[2]),
        pltpu.make_async_copy(W2_h, Wv.at[2], sems.at[3]),
        pltpu.make_async_copy(a0_h, av.at[0:1, :], sems.at[4]),
        pltpu.make_async_copy(a1_h, av.at[1:2, :], sems.at[5]),
        pltpu.make_async_copy(a2_h, av.at[2:3, :], sems.at[6]),
        pltpu.make_async_copy(Wo_h, Wov, sems.at[7]),
        pltpu.make_async_copy(ao_h, aov.at[0:1, :], sems.at[8]),
    ]
    mcps = [
        pltpu.make_async_copy(m_hbm.at[_BNDS[c]:_BNDS[c + 1], :],
                              mv.at[_BNDS[c]:_BNDS[c + 1], :], sems.at[9 + c])
        for c in range(NCHUNK)
    ]
    for c in cps + mcps:
        c.start()
    for c in cps:
        c.wait()

    x = xv[...]
    ones_col = jnp.ones((N, 1), jnp.float32)
    hps = []
    exps = []
    for k in range(NHEADS):
        h = jax.lax.dot_general(x, Wv[k], _DOT_T,
                                preferred_element_type=jnp.float32)
        ak = av[k, :]
        al = ak[:NHID].reshape(NHID, 1)
        ar = ak[NHID:].reshape(NHID, 1)
        ex = _node_exps(h, al, ar)
        exps.append(ex)
        hps.append((jnp.concatenate([h, ones_col], axis=1) * ex[2])
                   .astype(jnp.bfloat16))

    aggs = [[] for _ in range(NHEADS)]
    for c in range(NCHUNK):
        mcps[c].wait()
        rows = slice(_BNDS[c], _BNDS[c + 1])
        m16 = mv[rows, :].astype(jnp.bfloat16)
        m16v[rows, :] = m16
        for k in range(NHEADS):
            S = _masked_attn(exps[k], rows, m16)
            aggs[k].append(jnp.dot(S, hps[k],
                                   preferred_element_type=jnp.float32))
    heads = []
    for k in range(NHEADS):
        agg = jnp.concatenate(aggs[k], axis=0)
        rinv = 1.0 / agg[:, NHID:NHID + 1]
        heads.append(_elu(agg[:, :NHID] * rinv))
    hcat = jnp.concatenate(heads, axis=1)

    ho = jax.lax.dot_general(hcat, Wov[...], _DOT_T,
                             preferred_element_type=jnp.float32)
    ao = aov[0, :]
    exo = _node_exps(ho, ao[:NCLASS].reshape(NCLASS, 1),
                     ao[NCLASS:2 * NCLASS].reshape(NCLASS, 1))
    S = _masked_attn(exo, None, m16v[...])
    hop = ((jnp.concatenate([ho, ones_col], axis=1)) * exo[2]
           ).astype(jnp.bfloat16)
    agg = jnp.dot(S, hop, preferred_element_type=jnp.float32)
    aggT = agg.T
    rinv = 1.0 / aggT[NCLASS:NCLASS + 1, :]
    outT = _elu(aggT[:NCLASS, :] * rinv)
    mx = jnp.max(outT, axis=0, keepdims=True)
    zz = outT - mx
    out_ref[...] = zz - jnp.log(jnp.sum(jnp.exp(zz), axis=0, keepdims=True))


def kernel(x, adj, W0, W1, W2, a0, a1, a2, W_out, a_out):
    res = pl.pallas_call(
        _body,
        in_specs=[pl.BlockSpec(memory_space=pl.ANY)] * 10,
        out_shape=jax.ShapeDtypeStruct((NCLASS, N), jnp.float32),
        scratch_shapes=[
            pltpu.VMEM((N, NFEAT), jnp.float32),
            pltpu.VMEM((N, N), jnp.int32),
            pltpu.VMEM((N, N), jnp.bfloat16),
            pltpu.VMEM((NHEADS, NHID, NFEAT), jnp.float32),
            pltpu.VMEM((NHEADS, 2 * NHID), jnp.float32),
            pltpu.VMEM((NCLASS, NHID * NHEADS), jnp.float32),
            pltpu.VMEM((1, 2 * NCLASS), jnp.float32),
            pltpu.SemaphoreType.DMA((9 + NCHUNK,)),
        ],
    )(*[pltpu.with_memory_space_constraint(v, pltpu.MemorySpace.HBM)
        for v in (x, adj.astype(jnp.int32), W0.T, W1.T, W2.T, a0, a1, a2,
                  W_out.T, a_out)])
    return res.T


# speedup vs baseline: 1.1023x; 1.1023x over previous
import jax
import jax.numpy as jnp
from jax.experimental import pallas as pl
from jax.experimental.pallas import tpu as pltpu

ALPHA = 0.2
N = 1024
NFEAT = 256
NHID = 64
NHEADS = 3
NCLASS = 40
_BNDS = (0, 512, 1024)
NCHUNK = len(_BNDS) - 1

_DOT_T = (((1,), (1,)), ((), ()))


def _elu(x):
    return jnp.where(x >= 0, x, jnp.exp(x) - 1.0)


def _node_exps(h, al, ar):
    M = jnp.concatenate([(1.0 - ALPHA) * al, (1.0 - ALPHA) * ar, -ar],
                        axis=1)
    E = jnp.exp(jnp.dot(h, M, preferred_element_type=jnp.float32))
    b16 = jnp.bfloat16
    return (E[:, 0:1].astype(b16),
            E[:, 1:2].reshape(1, N).astype(b16),
            E[:, 2:3])


def _masked_attn(ex, rows, m16):
    r, E3, _ = ex
    if rows is not None:
        r = r[rows]
    return jnp.minimum(r * E3, m16)


def _body(x_hbm, m_hbm, W0_h, W1_h, W2_h, a0_h, a1_h, a2_h, Wo_h, ao_h,
          out_ref, xv, mv, m16v, Wv, av, Wov, aov, sems):
    cps = [
        pltpu.make_async_copy(x_hbm, xv, sems.at[0]),
        pltpu.make_async_copy(W0_h, Wv.at[0], sems.at[1]),
        pltpu.make_async_copy(W1_h, Wv.at[1], sems.at[2]),
        pltpu.make_async_copy(W2_h, Wv.at[2], sems.at[3]),
        pltpu.make_async_copy(a0_h, av.at[0:1, :], sems.at[4]),
        pltpu.make_async_copy(a1_h, av.at[1:2, :], sems.at[5]),
        pltpu.make_async_copy(a2_h, av.at[2:3, :], sems.at[6]),
        pltpu.make_async_copy(Wo_h, Wov, sems.at[7]),
        pltpu.make_async_copy(ao_h, aov.at[0:1, :], sems.at[8]),
    ]
    mcps = [
        pltpu.make_async_copy(m_hbm.at[_BNDS[c]:_BNDS[c + 1], :],
                              mv.at[_BNDS[c]:_BNDS[c + 1], :], sems.at[9 + c])
        for c in range(NCHUNK)
    ]
    for c in cps + mcps:
        c.start()
    for c in cps:
        c.wait()

    x = xv[...]
    ones_col = jnp.ones((N, 1), jnp.float32)
    hps = []
    exps = []
    for k in range(NHEADS):
        h = jax.lax.dot_general(x, Wv[k], _DOT_T,
                                preferred_element_type=jnp.float32)
        ak = av[k, :]
        al = ak[:NHID].reshape(NHID, 1)
        ar = ak[NHID:].reshape(NHID, 1)
        ex = _node_exps(h, al, ar)
        exps.append(ex)
        hps.append((jnp.concatenate([h, ones_col], axis=1) * ex[2])
                   .astype(jnp.bfloat16))

    aggs = [[] for _ in range(NHEADS)]
    for c in range(NCHUNK):
        mcps[c].wait()
        rows = slice(_BNDS[c], _BNDS[c + 1])
        m16 = mv[rows, :].astype(jnp.bfloat16)
        m16v[rows, :] = m16
        for k in range(NHEADS):
            S = _masked_attn(exps[k], rows, m16)
            aggs[k].append(jnp.dot(S, hps[k],
                                   preferred_element_type=jnp.float32))
    heads = []
    for k in range(NHEADS):
        agg = jnp.concatenate(aggs[k], axis=0)
        rinv = 1.0 / agg[:, NHID:NHID + 1]
        heads.append(_elu(agg[:, :NHID] * rinv))
    hcat = jnp.concatenate(heads, axis=1)

    ho = jax.lax.dot_general(hcat, Wov[...], _DOT_T,
                             preferred_element_type=jnp.float32)
    ao = aov[0, :]
    exo = _node_exps(ho, ao[:NCLASS].reshape(NCLASS, 1),
                     ao[NCLASS:2 * NCLASS].reshape(NCLASS, 1))
    S = _masked_attn(exo, None, m16v[...])
    hop = ((jnp.concatenate([ho, ones_col], axis=1)) * exo[2]
           ).astype(jnp.bfloat16)
    agg = jnp.dot(S, hop, preferred_element_type=jnp.float32)
    aggT = agg.T
    rinv = 1.0 / aggT[NCLASS:NCLASS + 1, :]
    outT = _elu(aggT[:NCLASS, :] * rinv)
    mx = jnp.max(outT, axis=0, keepdims=True)
    zz = outT - mx
    out_ref[...] = zz - jnp.log(jnp.sum(jnp.exp(zz), axis=0, keepdims=True))


def kernel(x, adj, W0, W1, W2, a0, a1, a2, W_out, a_out):
    res = pl.pallas_call(
        _body,
        in_specs=[pl.BlockSpec(memory_space=pl.ANY)] * 10,
        out_shape=jax.ShapeDtypeStruct((NCLASS, N), jnp.float32),
        scratch_shapes=[
            pltpu.VMEM((N, NFEAT), jnp.float32),
            pltpu.VMEM((N, N), jnp.int32),
            pltpu.VMEM((N, N), jnp.bfloat16),
            pltpu.VMEM((NHEADS, NHID, NFEAT), jnp.float32),
            pltpu.VMEM((NHEADS, 2 * NHID), jnp.float32),
            pltpu.VMEM((NCLASS, NHID * NHEADS), jnp.float32),
            pltpu.VMEM((1, 2 * NCLASS), jnp.float32),
            pltpu.SemaphoreType.DMA((9 + NCHUNK,)),
        ],
    )(*[pltpu.with_memory_space_constraint(v, pltpu.MemorySpace.HBM)
        for v in (x, adj.astype(jnp.int32), W0.T, W1.T, W2.T, a0, a1, a2,
                  W_out.T, a_out)])
    return res.T
